# baseline (speedup 1.0000x reference)
import functools

import jax
import jax.numpy as jnp
from jax import lax
from jax.experimental import pallas as pl
from jax.experimental.pallas import tpu as pltpu
from jax.experimental.pallas import tpu_sc as plsc

_L = 16
_G = 8


def _gather_body(num_cores, b_per_w, D, idx_hbm, tab_t_hbm, out_t_hbm,
                 idx_v, slab_a, slab_b, buf_v, sem_a, sem_b):
    wid = lax.axis_index("s") * num_cores + lax.axis_index("c")
    base = wid * b_per_w
    pltpu.sync_copy(idx_hbm.at[pl.ds(base, b_per_w)], idx_v.at[pl.ds(0, b_per_w)])

    lane_iota = lax.iota(jnp.int32, _L)
    kvec = lane_iota & (_G - 1)
    mask = lane_iota < _G

    def fire(g, slab, sem):
        col_vec = (idx_v[pl.ds(g * _G, _L)] >> 7) << 7
        for k in range(_G):
            col = pl.multiple_of(col_vec[k], 128)
            pltpu.async_copy(
                tab_t_hbm.at[:, pl.ds(col, 128)], slab.at[k], sem
            )

    def wait(slab, sem):
        for k in range(_G):
            pltpu.make_async_copy(
                tab_t_hbm.at[:, pl.ds(0, 128)], slab.at[k], sem
            ).wait()

    def extract(g, slab):
        i0 = g * _G
        lvec = idx_v[pl.ds(i0, _L)] & 127
        colidx = i0 + lane_iota
        for c in range(D):
            cvec = jnp.full((_L,), c, jnp.int32)
            vals = plsc.load_gather(slab, [kvec, cvec, lvec])
            plsc.store_scatter(buf_v, [cvec, colidx], vals, mask=mask)

    n_groups = b_per_w // _G
    fire(0, slab_a, sem_a)
    fire(1, slab_b, sem_b)

    def body(t, _):
        wait(slab_a, sem_a)
        extract(2 * t, slab_a)
        fire(2 * t + 2, slab_a, sem_a)
        wait(slab_b, sem_b)
        extract(2 * t + 1, slab_b)
        fire(2 * t + 3, slab_b, sem_b)
        return ()

    lax.fori_loop(0, n_groups // 2 - 1, body, (), unroll=False)
    wait(slab_a, sem_a)
    extract(n_groups - 2, slab_a)
    wait(slab_b, sem_b)
    extract(n_groups - 1, slab_b)

    pltpu.sync_copy(buf_v, out_t_hbm.at[:, pl.ds(base, b_per_w)])


@functools.cache
def _build(B, V, D):
    info = plsc.get_sparse_core_info()
    nw = info.num_cores * info.num_subcores
    assert B % (8 * nw) == 0
    b_per_w = B // nw
    mesh = plsc.VectorSubcoreMesh(core_axis_name="c", subcore_axis_name="s")
    return pl.kernel(
        functools.partial(_gather_body, info.num_cores, b_per_w, D),
        mesh=mesh,
        out_type=jax.ShapeDtypeStruct((D, B), jnp.float32),
        scratch_types=[
            pltpu.VMEM((b_per_w + _L,), jnp.int32),
            pltpu.VMEM((_G, D, 128), jnp.float32),
            pltpu.VMEM((_G, D, 128), jnp.float32),
            pltpu.VMEM((D, b_per_w), jnp.float32),
            pltpu.SemaphoreType.DMA,
            pltpu.SemaphoreType.DMA,
        ],
        compiler_params=pltpu.CompilerParams(needs_layout_passes=False),
    )


def kernel(context_ids, weight):
    B = context_ids.shape[0]
    V, D = weight.shape
    out_t = _build(B, V, D)(context_ids.astype(jnp.int32), weight.T)
    return out_t.T


# speedup vs baseline: 1.0063x; 1.0063x over previous
import functools

import jax
import jax.numpy as jnp
from jax import lax
from jax.experimental import pallas as pl
from jax.experimental.pallas import tpu as pltpu
from jax.experimental.pallas import tpu_sc as plsc

_L = 16
_G = 8


def _gather_body(num_cores, b_per_w, D, idx_hbm, tab_t_hbm, out_t_hbm,
                 idx_v, slab_a, slab_b, buf_v, sem_a, sem_b):
    wid = lax.axis_index("s") * num_cores + lax.axis_index("c")
    base = wid * b_per_w
    pltpu.sync_copy(idx_hbm.at[pl.ds(base, b_per_w)], idx_v.at[pl.ds(0, b_per_w)])

    lane_iota = lax.iota(jnp.int32, _L)
    kvec = lane_iota & (_G - 1)
    mask = lane_iota < _G

    def fire(g, slab, sem):
        col_vec = (idx_v[pl.ds(g * _G, _L)] >> 7) << 7
        for k in range(_G):
            col = pl.multiple_of(col_vec[k], 128)
            for tr in range(4):
                pltpu.async_copy(
                    tab_t_hbm.at[pl.ds(8 * tr, 8), pl.ds(col, 128)],
                    slab.at[k, pl.ds(8 * tr, 8)],
                    sem,
                )

    def wait(slab, sem):
        for k in range(_G):
            pltpu.make_async_copy(
                tab_t_hbm.at[:, pl.ds(0, 128)], slab.at[k], sem
            ).wait()

    def extract(g, slab):
        i0 = g * _G
        lvec = idx_v[pl.ds(i0, _L)] & 127
        colidx = i0 + lane_iota
        for c in range(D):
            cvec = jnp.full((_L,), c, jnp.int32)
            vals = plsc.load_gather(slab, [kvec, cvec, lvec])
            plsc.store_scatter(buf_v, [cvec, colidx], vals, mask=mask)

    n_groups = b_per_w // _G
    fire(0, slab_a, sem_a)
    fire(1, slab_b, sem_b)

    def body(t, _):
        wait(slab_a, sem_a)
        extract(2 * t, slab_a)
        fire(2 * t + 2, slab_a, sem_a)
        wait(slab_b, sem_b)
        extract(2 * t + 1, slab_b)
        fire(2 * t + 3, slab_b, sem_b)
        return ()

    lax.fori_loop(0, n_groups // 2 - 1, body, (), unroll=False)
    wait(slab_a, sem_a)
    extract(n_groups - 2, slab_a)
    wait(slab_b, sem_b)
    extract(n_groups - 1, slab_b)

    pltpu.sync_copy(buf_v, out_t_hbm.at[:, pl.ds(base, b_per_w)])


@functools.cache
def _build(B, V, D):
    info = plsc.get_sparse_core_info()
    nw = info.num_cores * info.num_subcores
    assert B % (8 * nw) == 0
    b_per_w = B // nw
    mesh = plsc.VectorSubcoreMesh(core_axis_name="c", subcore_axis_name="s")
    return pl.kernel(
        functools.partial(_gather_body, info.num_cores, b_per_w, D),
        mesh=mesh,
        out_type=jax.ShapeDtypeStruct((D, B), jnp.float32),
        scratch_types=[
            pltpu.VMEM((b_per_w + _L,), jnp.int32),
            pltpu.VMEM((_G, D, 128), jnp.float32),
            pltpu.VMEM((_G, D, 128), jnp.float32),
            pltpu.VMEM((D, b_per_w), jnp.float32),
            pltpu.SemaphoreType.DMA,
            pltpu.SemaphoreType.DMA,
        ],
        compiler_params=pltpu.CompilerParams(needs_layout_passes=False),
    )


def kernel(context_ids, weight):
    B = context_ids.shape[0]
    V, D = weight.shape
    out_t = _build(B, V, D)(context_ids.astype(jnp.int32), weight.T)
    return out_t.T
